# baseline (speedup 1.0000x reference)
import dataclasses
import functools

import jax
import jax.numpy as jnp
from jax import lax
from jax.experimental import pallas as pl
from jax.experimental.pallas import tpu as pltpu
from jax.experimental.pallas import tpu_sc as plsc

B, N, C, K = 2, 4096, 16, 16
RB = 256


def _topk_body(xr_ref, xa_ref, sqr_ref, sqa_ref, idx_ref):
    xr = xr_ref[0]
    xa = xa_ref[0]
    g = lax.dot_general(xr, xa, (((1,), (1,)), ((), ())),
                        preferred_element_type=jnp.float32)
    sqr = jnp.reshape(sqr_ref[0], (RB, 1))
    sqa = sqa_ref[0]
    d = jnp.maximum(sqr + sqa - 2.0 * g, 0.0)
    iota = lax.broadcasted_iota(jnp.int32, (RB, N), 1)
    cols = []
    for _ in range(K):
        m = jnp.min(d, axis=1, keepdims=True)
        sel = jnp.where(d == m, iota, jnp.int32(N))
        c = jnp.min(sel, axis=1, keepdims=True)
        cols.append(c)
        d = jnp.where(iota == c, jnp.float32(1e30), d)
    idx_ref[...] = jnp.concatenate(cols, axis=1)


def _tc_topk(x):
    sq = jnp.sum(x * x, axis=-1)
    sqr = sq.reshape(B * N // RB, 1, RB)
    sqa = sq.reshape(B, 1, N)
    return pl.pallas_call(
        _topk_body,
        grid=(B, N // RB),
        in_specs=[
            pl.BlockSpec((1, RB, C), lambda b, i: (b, i, 0)),
            pl.BlockSpec((1, N, C), lambda b, i: (b, 0, 0)),
            pl.BlockSpec((1, 1, RB), lambda b, i: (b * (N // RB) + i, 0, 0)),
            pl.BlockSpec((1, 1, N), lambda b, i: (b, 0, 0)),
        ],
        out_specs=pl.BlockSpec((RB, K), lambda b, i: (b * (N // RB) + i, 0)),
        out_shape=jax.ShapeDtypeStruct((B * N, K), jnp.int32),
        compiler_params=pltpu.CompilerParams(
            dimension_semantics=("parallel", "parallel")),
    )(x, x, sqr, sqa)


NW = 32
ROWS_PER_W = (B * N) // NW
CHUNK_ROWS = 4
CHUNK = CHUNK_ROWS * N
NCHUNK = ROWS_PER_W // CHUNK_ROWS
NBUF = 4


def _sc_body(idx_hbm, out_hbm, idx_v, b0, b1, b2, b3, s0, s1, s2, s3):
    bufs = [b0, b1, b2, b3]
    sems = [s0, s1, s2, s3]
    c = lax.axis_index("c")
    s = lax.axis_index("s")
    w = s * 2 + c
    base_row = w * ROWS_PER_W
    ones16 = jnp.ones((16,), jnp.float32)
    zeros16 = jnp.zeros((16,), jnp.float32)

    for q in range(NBUF):
        @pl.loop(0, CHUNK, step=16)
        def _(i, _b=bufs[q]):
            _b[pl.ds(i, 16)] = zeros16

    pltpu.sync_copy(idx_hbm.at[pl.ds(base_row, ROWS_PER_W)], idx_v)

    def do_chunk(ci, q):
        for r in range(CHUNK_ROWS):
            colv = idx_v[ci * CHUNK_ROWS + r, :]
            plsc.store_scatter(bufs[q], [colv + jnp.int32(r * N)], ones16)
        dst = out_hbm.at[pl.ds((base_row + ci * CHUNK_ROWS) * N, CHUNK)]
        pltpu.async_copy(bufs[q], dst, sems[q])

    def undo_chunk(ci, q):
        pltpu.make_async_copy(
            bufs[q], out_hbm.at[pl.ds((base_row + ci * CHUNK_ROWS) * N, CHUNK)],
            sems[q]).wait()
        for r in range(CHUNK_ROWS):
            colv = idx_v[ci * CHUNK_ROWS + r, :]
            plsc.store_scatter(bufs[q], [colv + jnp.int32(r * N)], zeros16)

    for q in range(NBUF):
        do_chunk(q, q)

    @pl.loop(NBUF, NCHUNK, step=NBUF)
    def _(ci):
        for q in range(NBUF):
            undo_chunk(ci + q - NBUF, q)
            do_chunk(ci + q, q)

    for q in range(NBUF):
        pltpu.make_async_copy(
            bufs[q],
            out_hbm.at[pl.ds((base_row + (NCHUNK - NBUF + q) * CHUNK_ROWS) * N,
                             CHUNK)],
            sems[q]).wait()


def _sc_adj(idx):
    mesh = plsc.VectorSubcoreMesh(core_axis_name="c", subcore_axis_name="s")
    cp = pltpu.CompilerParams()
    if "needs_layout_passes" in pltpu.CompilerParams.__dataclass_fields__:
        cp = dataclasses.replace(cp, needs_layout_passes=False)
    kern = pl.kernel(
        _sc_body,
        out_type=jax.ShapeDtypeStruct((B * N * N,), jnp.float32),
        mesh=mesh,
        scratch_types=[pltpu.VMEM((ROWS_PER_W, K), jnp.int32)]
        + [pltpu.VMEM((CHUNK,), jnp.float32)] * NBUF
        + [pltpu.SemaphoreType.DMA] * NBUF,
        compiler_params=cp,
    )
    return kern(idx)


def kernel(x):
    idx = _tc_topk(x)
    adj = _sc_adj(idx)
    return adj.reshape(B, N, N)


# speedup vs baseline: 1.0481x; 1.0481x over previous
import dataclasses
import functools

import jax
import jax.numpy as jnp
from jax import lax
from jax.experimental import pallas as pl
from jax.experimental.pallas import tpu as pltpu
from jax.experimental.pallas import tpu_sc as plsc

B, N, C, K = 2, 4096, 16, 16
RB = 256


def _topk_body(xr_ref, xa_ref, sqr_ref, sqa_ref, idx_ref):
    xr = xr_ref[0]
    xa = xa_ref[0]
    g = lax.dot_general(xr, xa, (((1,), (1,)), ((), ())),
                        preferred_element_type=jnp.float32)
    sqr = jnp.reshape(sqr_ref[0], (RB, 1))
    sqa = sqa_ref[0]
    d = jnp.maximum(sqr + sqa - 2.0 * g, 0.0)
    iota = lax.broadcasted_iota(jnp.int32, (RB, N), 1).astype(jnp.float32)
    cols = []
    for _ in range(K):
        v, i = d, iota
        w = N
        while w > 128:
            h = w // 2
            take = v[:, h:] < v[:, :h]
            v = jnp.where(take, v[:, h:], v[:, :h])
            i = jnp.where(take, i[:, h:], i[:, :h])
            w = h
        m = jnp.min(v, axis=1, keepdims=True)
        c = jnp.min(jnp.where(v == m, i, jnp.float32(N)), axis=1,
                    keepdims=True)
        cols.append(c)
        d = jnp.where(iota == c, jnp.float32(1e30), d)
    idx_ref[...] = jnp.concatenate(cols, axis=1).astype(jnp.int32)


def _tc_topk(x):
    sq = jnp.sum(x * x, axis=-1)
    sqr = sq.reshape(B * N // RB, 1, RB)
    sqa = sq.reshape(B, 1, N)
    return pl.pallas_call(
        _topk_body,
        grid=(B, N // RB),
        in_specs=[
            pl.BlockSpec((1, RB, C), lambda b, i: (b, i, 0)),
            pl.BlockSpec((1, N, C), lambda b, i: (b, 0, 0)),
            pl.BlockSpec((1, 1, RB), lambda b, i: (b * (N // RB) + i, 0, 0)),
            pl.BlockSpec((1, 1, N), lambda b, i: (b, 0, 0)),
        ],
        out_specs=pl.BlockSpec((RB, K), lambda b, i: (b * (N // RB) + i, 0)),
        out_shape=jax.ShapeDtypeStruct((B * N, K), jnp.int32),
        compiler_params=pltpu.CompilerParams(
            dimension_semantics=("parallel", "parallel")),
    )(x, x, sqr, sqa)


NW = 32
ROWS_PER_W = (B * N) // NW
CHUNK_ROWS = 4
CHUNK = CHUNK_ROWS * N
NCHUNK = ROWS_PER_W // CHUNK_ROWS
NBUF = 4


def _sc_body(idx_hbm, out_hbm, idx_v, b0, b1, b2, b3, s0, s1, s2, s3):
    bufs = [b0, b1, b2, b3]
    sems = [s0, s1, s2, s3]
    c = lax.axis_index("c")
    s = lax.axis_index("s")
    w = s * 2 + c
    base_row = w * ROWS_PER_W
    ones16 = jnp.ones((16,), jnp.float32)
    zeros16 = jnp.zeros((16,), jnp.float32)

    for q in range(NBUF):
        @pl.loop(0, CHUNK, step=16)
        def _(i, _b=bufs[q]):
            _b[pl.ds(i, 16)] = zeros16

    pltpu.sync_copy(idx_hbm.at[pl.ds(base_row, ROWS_PER_W)], idx_v)

    def do_chunk(ci, q):
        for r in range(CHUNK_ROWS):
            colv = idx_v[ci * CHUNK_ROWS + r, :]
            plsc.store_scatter(bufs[q], [colv + jnp.int32(r * N)], ones16)
        dst = out_hbm.at[pl.ds((base_row + ci * CHUNK_ROWS) * N, CHUNK)]
        pltpu.async_copy(bufs[q], dst, sems[q])

    def undo_chunk(ci, q):
        pltpu.make_async_copy(
            bufs[q], out_hbm.at[pl.ds((base_row + ci * CHUNK_ROWS) * N, CHUNK)],
            sems[q]).wait()
        for r in range(CHUNK_ROWS):
            colv = idx_v[ci * CHUNK_ROWS + r, :]
            plsc.store_scatter(bufs[q], [colv + jnp.int32(r * N)], zeros16)

    for q in range(NBUF):
        do_chunk(q, q)

    @pl.loop(NBUF, NCHUNK, step=NBUF)
    def _(ci):
        for q in range(NBUF):
            undo_chunk(ci + q - NBUF, q)
            do_chunk(ci + q, q)

    for q in range(NBUF):
        pltpu.make_async_copy(
            bufs[q],
            out_hbm.at[pl.ds((base_row + (NCHUNK - NBUF + q) * CHUNK_ROWS) * N,
                             CHUNK)],
            sems[q]).wait()


def _sc_adj(idx):
    mesh = plsc.VectorSubcoreMesh(core_axis_name="c", subcore_axis_name="s")
    cp = pltpu.CompilerParams()
    if "needs_layout_passes" in pltpu.CompilerParams.__dataclass_fields__:
        cp = dataclasses.replace(cp, needs_layout_passes=False)
    kern = pl.kernel(
        _sc_body,
        out_type=jax.ShapeDtypeStruct((B * N * N,), jnp.float32),
        mesh=mesh,
        scratch_types=[pltpu.VMEM((ROWS_PER_W, K), jnp.int32)]
        + [pltpu.VMEM((CHUNK,), jnp.float32)] * NBUF
        + [pltpu.SemaphoreType.DMA] * NBUF,
        compiler_params=cp,
    )
    return kern(idx)


def kernel(x):
    idx = _tc_topk(x)
    adj = _sc_adj(idx)
    return adj.reshape(B, N, N)


# speedup vs baseline: 1.0509x; 1.0027x over previous
import dataclasses
import functools

import jax
import jax.numpy as jnp
from jax import lax
from jax.experimental import pallas as pl
from jax.experimental.pallas import tpu as pltpu
from jax.experimental.pallas import tpu_sc as plsc

B, N, C, K = 2, 4096, 16, 16
RB = 512


def _topk_body(xr_ref, xa_ref, sqr_ref, sqa_ref, idx_ref):
    xr = xr_ref[0]
    xa = xa_ref[0]
    g = lax.dot_general(xr, xa, (((1,), (1,)), ((), ())),
                        preferred_element_type=jnp.float32)
    sqr = jnp.reshape(sqr_ref[0], (RB, 1))
    sqa = sqa_ref[0]
    d = jnp.maximum(sqr + sqa - 2.0 * g, 0.0)
    iota = lax.broadcasted_iota(jnp.int32, (RB, N), 1).astype(jnp.float32)
    cols = []
    for _ in range(K):
        v, i = d, iota
        w = N
        while w > 128:
            h = w // 2
            take = v[:, h:] < v[:, :h]
            v = jnp.where(take, v[:, h:], v[:, :h])
            i = jnp.where(take, i[:, h:], i[:, :h])
            w = h
        m = jnp.min(v, axis=1, keepdims=True)
        c = jnp.min(jnp.where(v == m, i, jnp.float32(N)), axis=1,
                    keepdims=True)
        cols.append(c)
        d = jnp.where(iota == c, jnp.float32(1e30), d)
    idx_ref[...] = jnp.concatenate(cols, axis=1).astype(jnp.int32)


def _tc_topk(x):
    sq = jnp.sum(x * x, axis=-1)
    sqr = sq.reshape(B * N // RB, 1, RB)
    sqa = sq.reshape(B, 1, N)
    return pl.pallas_call(
        _topk_body,
        grid=(B, N // RB),
        in_specs=[
            pl.BlockSpec((1, RB, C), lambda b, i: (b, i, 0)),
            pl.BlockSpec((1, N, C), lambda b, i: (b, 0, 0)),
            pl.BlockSpec((1, 1, RB), lambda b, i: (b * (N // RB) + i, 0, 0)),
            pl.BlockSpec((1, 1, N), lambda b, i: (b, 0, 0)),
        ],
        out_specs=pl.BlockSpec((RB, K), lambda b, i: (b * (N // RB) + i, 0)),
        out_shape=jax.ShapeDtypeStruct((B * N, K), jnp.int32),
        compiler_params=pltpu.CompilerParams(
            dimension_semantics=("parallel", "parallel")),
    )(x, x, sqr, sqa)


NW = 32
ROWS_PER_W = (B * N) // NW
CHUNK_ROWS = 4
CHUNK = CHUNK_ROWS * N
NCHUNK = ROWS_PER_W // CHUNK_ROWS
NBUF = 4


def _sc_body(idx_hbm, out_hbm, idx_v, b0, b1, b2, b3, s0, s1, s2, s3):
    bufs = [b0, b1, b2, b3]
    sems = [s0, s1, s2, s3]
    c = lax.axis_index("c")
    s = lax.axis_index("s")
    w = s * 2 + c
    base_row = w * ROWS_PER_W
    ones16 = jnp.ones((16,), jnp.float32)
    zeros16 = jnp.zeros((16,), jnp.float32)

    for q in range(NBUF):
        @pl.loop(0, CHUNK, step=16)
        def _(i, _b=bufs[q]):
            _b[pl.ds(i, 16)] = zeros16

    pltpu.sync_copy(idx_hbm.at[pl.ds(base_row, ROWS_PER_W)], idx_v)

    def do_chunk(ci, q):
        for r in range(CHUNK_ROWS):
            colv = idx_v[ci * CHUNK_ROWS + r, :]
            plsc.store_scatter(bufs[q], [colv + jnp.int32(r * N)], ones16)
        dst = out_hbm.at[pl.ds((base_row + ci * CHUNK_ROWS) * N, CHUNK)]
        pltpu.async_copy(bufs[q], dst, sems[q])

    def undo_chunk(ci, q):
        pltpu.make_async_copy(
            bufs[q], out_hbm.at[pl.ds((base_row + ci * CHUNK_ROWS) * N, CHUNK)],
            sems[q]).wait()
        for r in range(CHUNK_ROWS):
            colv = idx_v[ci * CHUNK_ROWS + r, :]
            plsc.store_scatter(bufs[q], [colv + jnp.int32(r * N)], zeros16)

    for q in range(NBUF):
        do_chunk(q, q)

    @pl.loop(NBUF, NCHUNK, step=NBUF)
    def _(ci):
        for q in range(NBUF):
            undo_chunk(ci + q - NBUF, q)
            do_chunk(ci + q, q)

    for q in range(NBUF):
        pltpu.make_async_copy(
            bufs[q],
            out_hbm.at[pl.ds((base_row + (NCHUNK - NBUF + q) * CHUNK_ROWS) * N,
                             CHUNK)],
            sems[q]).wait()


def _sc_adj(idx):
    mesh = plsc.VectorSubcoreMesh(core_axis_name="c", subcore_axis_name="s")
    cp = pltpu.CompilerParams()
    if "needs_layout_passes" in pltpu.CompilerParams.__dataclass_fields__:
        cp = dataclasses.replace(cp, needs_layout_passes=False)
    kern = pl.kernel(
        _sc_body,
        out_type=jax.ShapeDtypeStruct((B * N * N,), jnp.float32),
        mesh=mesh,
        scratch_types=[pltpu.VMEM((ROWS_PER_W, K), jnp.int32)]
        + [pltpu.VMEM((CHUNK,), jnp.float32)] * NBUF
        + [pltpu.SemaphoreType.DMA] * NBUF,
        compiler_params=cp,
    )
    return kern(idx)


def kernel(x):
    idx = _tc_topk(x)
    adj = _sc_adj(idx)
    return adj.reshape(B, N, N)


# speedup vs baseline: 1.1971x; 1.1391x over previous
import dataclasses
import functools

import jax
import jax.numpy as jnp
from jax import lax
from jax.experimental import pallas as pl
from jax.experimental.pallas import tpu as pltpu
from jax.experimental.pallas import tpu_sc as plsc

B, N, C, K = 2, 4096, 16, 16
RB = 512


def _topk_body(xr_ref, xa_ref, sqr_ref, sqa_ref, idx_ref):
    xr = xr_ref[0]
    xa = xa_ref[0]
    g = lax.dot_general(xr, xa, (((1,), (1,)), ((), ())),
                        preferred_element_type=jnp.float32)
    sqr = jnp.reshape(sqr_ref[0], (RB, 1))
    sqa = sqa_ref[0]
    d = jnp.maximum(sqr + sqa - 2.0 * g, 0.0)
    iota = lax.broadcasted_iota(jnp.int32, (RB, N), 1).astype(jnp.float32)
    cols = []
    for _ in range(K):
        m = jnp.min(d, axis=1, keepdims=True)
        c = jnp.min(jnp.where(d == m, iota, jnp.float32(N)), axis=1,
                    keepdims=True)
        cols.append(c)
        d = jnp.where(iota == c, jnp.float32(1e30), d)
    idx_ref[...] = jnp.concatenate(cols, axis=1).astype(jnp.int32)


def _tc_topk(x):
    sq = jnp.sum(x * x, axis=-1)
    sqr = sq.reshape(B * N // RB, 1, RB)
    sqa = sq.reshape(B, 1, N)
    return pl.pallas_call(
        _topk_body,
        grid=(B, N // RB),
        in_specs=[
            pl.BlockSpec((1, RB, C), lambda b, i: (b, i, 0)),
            pl.BlockSpec((1, N, C), lambda b, i: (b, 0, 0)),
            pl.BlockSpec((1, 1, RB), lambda b, i: (b * (N // RB) + i, 0, 0)),
            pl.BlockSpec((1, 1, N), lambda b, i: (b, 0, 0)),
        ],
        out_specs=pl.BlockSpec((RB, K), lambda b, i: (b * (N // RB) + i, 0)),
        out_shape=jax.ShapeDtypeStruct((B * N, K), jnp.int32),
        compiler_params=pltpu.CompilerParams(
            dimension_semantics=("parallel", "parallel")),
    )(x, x, sqr, sqa)


NW = 32
ROWS_PER_W = (B * N) // NW
CHUNK_ROWS = 4
CHUNK = CHUNK_ROWS * N
NCHUNK = ROWS_PER_W // CHUNK_ROWS
NBUF = 4


def _sc_body(idx_hbm, out_hbm, idx_v, b0, b1, b2, b3, s0, s1, s2, s3):
    bufs = [b0, b1, b2, b3]
    sems = [s0, s1, s2, s3]
    c = lax.axis_index("c")
    s = lax.axis_index("s")
    w = s * 2 + c
    base_row = w * ROWS_PER_W
    ones16 = jnp.ones((16,), jnp.float32)
    zeros16 = jnp.zeros((16,), jnp.float32)

    for q in range(NBUF):
        @pl.loop(0, CHUNK, step=16)
        def _(i, _b=bufs[q]):
            _b[pl.ds(i, 16)] = zeros16

    pltpu.sync_copy(idx_hbm.at[pl.ds(base_row, ROWS_PER_W)], idx_v)

    def do_chunk(ci, q):
        for r in range(CHUNK_ROWS):
            colv = idx_v[ci * CHUNK_ROWS + r, :]
            plsc.store_scatter(bufs[q], [colv + jnp.int32(r * N)], ones16)
        dst = out_hbm.at[pl.ds((base_row + ci * CHUNK_ROWS) * N, CHUNK)]
        pltpu.async_copy(bufs[q], dst, sems[q])

    def undo_chunk(ci, q):
        pltpu.make_async_copy(
            bufs[q], out_hbm.at[pl.ds((base_row + ci * CHUNK_ROWS) * N, CHUNK)],
            sems[q]).wait()
        for r in range(CHUNK_ROWS):
            colv = idx_v[ci * CHUNK_ROWS + r, :]
            plsc.store_scatter(bufs[q], [colv + jnp.int32(r * N)], zeros16)

    for q in range(NBUF):
        do_chunk(q, q)

    @pl.loop(NBUF, NCHUNK, step=NBUF)
    def _(ci):
        for q in range(NBUF):
            undo_chunk(ci + q - NBUF, q)
            do_chunk(ci + q, q)

    for q in range(NBUF):
        pltpu.make_async_copy(
            bufs[q],
            out_hbm.at[pl.ds((base_row + (NCHUNK - NBUF + q) * CHUNK_ROWS) * N,
                             CHUNK)],
            sems[q]).wait()


def _sc_adj(idx):
    mesh = plsc.VectorSubcoreMesh(core_axis_name="c", subcore_axis_name="s")
    cp = pltpu.CompilerParams()
    if "needs_layout_passes" in pltpu.CompilerParams.__dataclass_fields__:
        cp = dataclasses.replace(cp, needs_layout_passes=False)
    kern = pl.kernel(
        _sc_body,
        out_type=jax.ShapeDtypeStruct((B * N * N,), jnp.float32),
        mesh=mesh,
        scratch_types=[pltpu.VMEM((ROWS_PER_W, K), jnp.int32)]
        + [pltpu.VMEM((CHUNK,), jnp.float32)] * NBUF
        + [pltpu.SemaphoreType.DMA] * NBUF,
        compiler_params=cp,
    )
    return kern(idx)


def kernel(x):
    idx = _tc_topk(x)
    adj = _sc_adj(idx)
    return adj.reshape(B, N, N)
